# Initial kernel scaffold; baseline (speedup 1.0000x reference)
import jax
import jax.numpy as jnp
from jax.experimental import pallas as pl


def kernel(q, k, v, W_perf, enc_w, enc_b, ln_g, ln_b, dec_w, dec_b):
    raise NotImplementedError("write your pallas kernel here")


# speedup vs baseline: 30.8054x; 30.8054x over previous
import functools
import math

import jax
import jax.numpy as jnp
from jax.experimental import pallas as pl

B, H, T, DH = 1, 16, 2048, 64
NB_FEAT = int(DH * math.log(DH))
MPAD = 384
PRED_LEN = 128
NSEL = 4
GROUP = T // PRED_LEN
TQ = 256


def _dotbf(a, b, dims):
    return jax.lax.dot_general(
        a.astype(jnp.bfloat16), b.astype(jnp.bfloat16),
        (dims, ((), ())), preferred_element_type=jnp.float32)


def _estimator_kernel(q_ref, k_ref, v_ref, w_ref, encw_ref, encb_ref,
                      lng_ref, lnb_ref, decw_ref, decb_ref, mask_ref):
    q = q_ref[0]
    k = k_ref[0]
    v = v_ref[0]
    w = w_ref[...]
    f32 = jnp.float32

    data_norm = DH ** -0.25
    ratio = NB_FEAT ** -0.5
    feat_ok = jax.lax.broadcasted_iota(jnp.int32, (1, MPAD), 1) < NB_FEAT

    projq = data_norm * _dotbf(q, w, ((1,), (1,)))
    projk = data_norm * _dotbf(k, w, ((1,), (1,)))

    diag_q = (data_norm ** 2) * jnp.sum(q * q, axis=-1, keepdims=True) * 0.5
    diag_k = (data_norm ** 2) * jnp.sum(k * k, axis=-1, keepdims=True) * 0.5
    neg = jnp.float32(-jnp.inf)
    stab_q = jnp.max(jnp.where(feat_ok, projq, neg), axis=-1, keepdims=True)
    stab_k = jnp.max(jnp.where(feat_ok, projk, neg))

    pq = ratio * (jnp.exp(projq - diag_q - stab_q) + 1e-6)
    pq = jnp.where(feat_ok, pq, 0.0)
    pk = ratio * (jnp.exp(projk - diag_k - stab_k) + 1e-6)
    pk = jnp.where(feat_ok, pk, 0.0)

    kv = _dotbf(pk, v, ((0,), (0,)))
    num = _dotbf(pq, kv, ((1,), (0,)))
    pk_sum = jnp.sum(pk, axis=0, keepdims=True)
    den = jnp.sum(pq.astype(jnp.bfloat16).astype(f32)
                  * pk_sum.astype(jnp.bfloat16).astype(f32),
                  axis=-1, keepdims=True)
    ctx = num / (den + 1e-6)

    encw = encw_ref[...]
    h = (_dotbf(v, encw[0:DH], ((1,), (0,)))
         + _dotbf(ctx, encw[DH:2 * DH], ((1,), (0,)))
         + _dotbf(v * ctx, encw[2 * DH:3 * DH], ((1,), (0,)))
         + encb_ref[...])
    mu = jnp.mean(h, axis=-1, keepdims=True)
    var = jnp.mean((h - mu) * (h - mu), axis=-1, keepdims=True)
    h = (h - mu) * jax.lax.rsqrt(var + 1e-5) * lng_ref[...] + lnb_ref[...]
    h = jax.nn.gelu(h)
    lowres = _dotbf(h, decw_ref[...], ((1,), (0,))) + decb_ref[...]

    col = jax.lax.broadcasted_iota(jnp.int32, (T, PRED_LEN), 1)
    x = lowres
    m = jnp.max(x, axis=-1, keepdims=True)
    for _ in range(NSEL - 1):
        first = jnp.min(jnp.where(x >= m, col, PRED_LEN), axis=-1, keepdims=True)
        x = jnp.where(col == first, neg, x)
        m = jnp.max(x, axis=-1, keepdims=True)
    mask_ref[0] = jnp.where(lowres >= m, 1.0, 0.0).astype(f32)


def _attention_kernel(q_ref, k_ref, v_ref, mask_ref, expand_ref, o_ref):
    q = q_ref[0]
    k = k_ref[0]
    v = v_ref[0]
    gmask = mask_ref[0]
    expand = expand_ref[...]

    s = _dotbf(q, k, ((1,), (1,))) * (DH ** -0.5)
    m = _dotbf(gmask, expand, ((1,), (0,)))
    s = jnp.where(m > 0.5, s, jnp.float32(-1e9))
    smax = jnp.max(s, axis=-1, keepdims=True)
    p = jnp.exp(s - smax)
    p = p / jnp.sum(p, axis=-1, keepdims=True)
    o_ref[0] = _dotbf(p, v, ((1,), (0,)))


@functools.partial(jax.jit, static_argnames=("interpret",))
def kernel(q, k, v, W_perf, enc_w, enc_b, ln_g, ln_b, dec_w, dec_b,
           interpret=False):
    f32 = jnp.float32
    q3 = q.reshape(H, T, DH)
    k3 = k.reshape(H, T, DH)
    v3 = v.reshape(H, T, DH)
    w_pad = jnp.zeros((MPAD, DH), f32).at[:NB_FEAT].set(W_perf)
    encb = enc_b.reshape(1, 2 * DH)
    lng = ln_g.reshape(1, 2 * DH)
    lnb = ln_b.reshape(1, 2 * DH)
    decb = dec_b.reshape(1, PRED_LEN)

    head_spec = pl.BlockSpec((1, T, DH), lambda h: (h, 0, 0))
    full = lambda shape: pl.BlockSpec(shape, lambda h: tuple(0 for _ in shape))

    gmask = pl.pallas_call(
        _estimator_kernel,
        grid=(H,),
        in_specs=[head_spec, head_spec, head_spec,
                  full((MPAD, DH)), full((3 * DH, 2 * DH)), full((1, 2 * DH)),
                  full((1, 2 * DH)), full((1, 2 * DH)),
                  full((2 * DH, PRED_LEN)), full((1, PRED_LEN))],
        out_specs=pl.BlockSpec((1, T, PRED_LEN), lambda h: (h, 0, 0)),
        out_shape=jax.ShapeDtypeStruct((H, T, PRED_LEN), f32),
        interpret=interpret,
    )(q3, k3, v3, w_pad, enc_w, encb, lng, lnb, dec_w, decb)

    gid = (jnp.arange(T, dtype=jnp.int32) * PRED_LEN) // T
    expand = (gid[None, :] == jnp.arange(PRED_LEN, dtype=jnp.int32)[:, None]
              ).astype(f32)

    out = pl.pallas_call(
        _attention_kernel,
        grid=(H, T // TQ),
        in_specs=[pl.BlockSpec((1, TQ, DH), lambda h, i: (h, i, 0)),
                  pl.BlockSpec((1, T, DH), lambda h, i: (h, 0, 0)),
                  pl.BlockSpec((1, T, DH), lambda h, i: (h, 0, 0)),
                  pl.BlockSpec((1, TQ, PRED_LEN), lambda h, i: (h, i, 0)),
                  pl.BlockSpec((PRED_LEN, T), lambda h, i: (0, 0))],
        out_specs=pl.BlockSpec((1, TQ, DH), lambda h, i: (h, i, 0)),
        out_shape=jax.ShapeDtypeStruct((H, T, DH), f32),
        interpret=interpret,
    )(q3, k3, v3, gmask, expand)

    return out.reshape(B, H, T, DH)


# speedup vs baseline: 30.8184x; 1.0004x over previous
import functools
import math

import jax
import jax.numpy as jnp
import numpy as np
from jax.experimental import pallas as pl
from jax.experimental.pallas import tpu as pltpu

B, H, T, DH = 1, 16, 2048, 64
NB_FEAT = int(DH * math.log(DH))
MPAD = 384
PRED_LEN = 128
NSEL = 4
GROUP = T // PRED_LEN
TQ = 256


def _dotbf(a, b, dims):
    return jax.lax.dot_general(
        a.astype(jnp.bfloat16), b.astype(jnp.bfloat16),
        (dims, ((), ())), preferred_element_type=jnp.float32)


def _estimator_kernel(q_ref, k_ref, v_ref, w_ref, encw_ref, encb_ref,
                      lng_ref, lnb_ref, decw_ref, decb_ref, mask_ref):
    q = q_ref[0]
    k = k_ref[0]
    v = v_ref[0]
    w = w_ref[...]
    f32 = jnp.float32

    data_norm = DH ** -0.25
    ratio = NB_FEAT ** -0.5
    feat_ok = jax.lax.broadcasted_iota(jnp.int32, (1, MPAD), 1) < NB_FEAT

    projq = data_norm * _dotbf(q, w, ((1,), (1,)))
    projk = data_norm * _dotbf(k, w, ((1,), (1,)))

    diag_q = (data_norm ** 2) * jnp.sum(q * q, axis=-1, keepdims=True) * 0.5
    diag_k = (data_norm ** 2) * jnp.sum(k * k, axis=-1, keepdims=True) * 0.5
    neg = jnp.float32(-jnp.inf)
    stab_q = jnp.max(jnp.where(feat_ok, projq, neg), axis=-1, keepdims=True)
    stab_k = jnp.max(jnp.where(feat_ok, projk, neg))

    pq = ratio * (jnp.exp(projq - diag_q - stab_q) + 1e-6)
    pq = jnp.where(feat_ok, pq, 0.0)
    pk = ratio * (jnp.exp(projk - diag_k - stab_k) + 1e-6)
    pk = jnp.where(feat_ok, pk, 0.0)

    kv = _dotbf(pk, v, ((0,), (0,)))
    num = _dotbf(pq, kv, ((1,), (0,)))
    pk_sum = jnp.sum(pk, axis=0, keepdims=True)
    den = jnp.sum(pq.astype(jnp.bfloat16).astype(f32)
                  * pk_sum.astype(jnp.bfloat16).astype(f32),
                  axis=-1, keepdims=True)
    ctx = num / (den + 1e-6)

    encw = encw_ref[...]
    h = (_dotbf(v, encw[0:DH], ((1,), (0,)))
         + _dotbf(ctx, encw[DH:2 * DH], ((1,), (0,)))
         + _dotbf(v * ctx, encw[2 * DH:3 * DH], ((1,), (0,)))
         + encb_ref[...])
    mu = jnp.mean(h, axis=-1, keepdims=True)
    var = jnp.mean((h - mu) * (h - mu), axis=-1, keepdims=True)
    h = (h - mu) * jax.lax.rsqrt(var + 1e-5) * lng_ref[...] + lnb_ref[...]
    h = jax.nn.gelu(h)
    lowres = _dotbf(h, decw_ref[...], ((1,), (0,))) + decb_ref[...]

    col = jax.lax.broadcasted_iota(jnp.int32, (T, PRED_LEN), 1)
    x = lowres
    m = jnp.max(x, axis=-1, keepdims=True)
    for _ in range(NSEL - 1):
        first = jnp.min(jnp.where(x >= m, col, PRED_LEN), axis=-1, keepdims=True)
        x = jnp.where(col == first, neg, x)
        m = jnp.max(x, axis=-1, keepdims=True)
    mask_ref[0] = jnp.where(lowres >= m, 1.0, 0.0).astype(f32)


def _attention_kernel(q_ref, k_ref, v_ref, mask_ref, expand_ref, o_ref):
    q = q_ref[0]
    k = k_ref[0]
    v = v_ref[0]
    gmask = mask_ref[0]
    expand = expand_ref[...]

    s = _dotbf(q, k, ((1,), (1,))) * (DH ** -0.5)
    m = _dotbf(gmask, expand, ((1,), (0,)))
    s = jnp.where(m > 0.5, s, jnp.float32(-1e9))
    smax = jnp.max(s, axis=-1, keepdims=True)
    p = jnp.exp(s - smax)
    p = p / jnp.sum(p, axis=-1, keepdims=True)
    o_ref[0] = _dotbf(p, v, ((1,), (0,)))


@functools.partial(jax.jit, static_argnames=("interpret",))
def kernel(q, k, v, W_perf, enc_w, enc_b, ln_g, ln_b, dec_w, dec_b,
           interpret=False):
    f32 = jnp.float32
    q3 = q.reshape(H, T, DH)
    k3 = k.reshape(H, T, DH)
    v3 = v.reshape(H, T, DH)
    w_pad = jnp.zeros((MPAD, DH), f32).at[:NB_FEAT].set(W_perf)
    encb = enc_b.reshape(1, 2 * DH)
    lng = ln_g.reshape(1, 2 * DH)
    lnb = ln_b.reshape(1, 2 * DH)
    decb = dec_b.reshape(1, PRED_LEN)

    head_spec = pl.BlockSpec((1, T, DH), lambda h: (h, 0, 0))
    full = lambda shape: pl.BlockSpec(shape, lambda h: tuple(0 for _ in shape))

    gmask = pl.pallas_call(
        _estimator_kernel,
        grid=(H,),
        in_specs=[head_spec, head_spec, head_spec,
                  full((MPAD, DH)), full((3 * DH, 2 * DH)), full((1, 2 * DH)),
                  full((1, 2 * DH)), full((1, 2 * DH)),
                  full((2 * DH, PRED_LEN)), full((1, PRED_LEN))],
        out_specs=pl.BlockSpec((1, T, PRED_LEN), lambda h: (h, 0, 0)),
        out_shape=jax.ShapeDtypeStruct((H, T, PRED_LEN), f32),
        compiler_params=pltpu.CompilerParams(
            dimension_semantics=("parallel",)),
        interpret=interpret,
    )(q3, k3, v3, w_pad, enc_w, encb, lng, lnb, dec_w, decb)

    gid_np = (np.arange(T) * PRED_LEN) // T
    expand = jnp.asarray(
        (gid_np[None, :] == np.arange(PRED_LEN)[:, None]).astype(np.float32))

    out = pl.pallas_call(
        _attention_kernel,
        grid=(H, T // TQ),
        in_specs=[pl.BlockSpec((1, TQ, DH), lambda h, i: (h, i, 0)),
                  pl.BlockSpec((1, T, DH), lambda h, i: (h, 0, 0)),
                  pl.BlockSpec((1, T, DH), lambda h, i: (h, 0, 0)),
                  pl.BlockSpec((1, TQ, PRED_LEN), lambda h, i: (h, i, 0)),
                  pl.BlockSpec((PRED_LEN, T), lambda h, i: (0, 0))],
        out_specs=pl.BlockSpec((1, TQ, DH), lambda h, i: (h, i, 0)),
        out_shape=jax.ShapeDtypeStruct((H, T, DH), f32),
        compiler_params=pltpu.CompilerParams(
            dimension_semantics=("parallel", "parallel")),
        interpret=interpret,
    )(q3, k3, v3, gmask, expand)

    return out.reshape(B, H, T, DH)


# speedup vs baseline: 34.8355x; 1.1303x over previous
import functools
import math

import jax
import jax.numpy as jnp
import numpy as np
from jax.experimental import pallas as pl
from jax.experimental.pallas import tpu as pltpu

B, H, T, DH = 1, 16, 2048, 64
NB_FEAT = int(DH * math.log(DH))
MPAD = 384
PRED_LEN = 128
NSEL = 4
TQ = 256


def _dotbf(a, b, dims):
    return jax.lax.dot_general(
        a.astype(jnp.bfloat16), b.astype(jnp.bfloat16),
        (dims, ((), ())), preferred_element_type=jnp.float32)


def _fused_kernel(q_ref, k_ref, v_ref, w_ref, encw_ref, encb_ref,
                  lng_ref, lnb_ref, decw_ref, decb_ref, expand_ref, o_ref):
    q = q_ref[0, 0]
    k = k_ref[0, 0]
    v = v_ref[0, 0]
    w = w_ref[...]
    f32 = jnp.float32

    data_norm = DH ** -0.25
    ratio = NB_FEAT ** -0.5
    feat_ok = jax.lax.broadcasted_iota(jnp.int32, (1, MPAD), 1) < NB_FEAT

    projq = data_norm * _dotbf(q, w, ((1,), (1,)))
    projk = data_norm * _dotbf(k, w, ((1,), (1,)))

    diag_q = (data_norm ** 2) * jnp.sum(q * q, axis=-1, keepdims=True) * 0.5
    diag_k = (data_norm ** 2) * jnp.sum(k * k, axis=-1, keepdims=True) * 0.5
    neg = jnp.float32(-jnp.inf)
    stab_q = jnp.max(jnp.where(feat_ok, projq, neg), axis=-1, keepdims=True)
    stab_k = jnp.max(jnp.where(feat_ok, projk, neg))

    pq = ratio * (jnp.exp(projq - diag_q - stab_q) + 1e-6)
    pq = jnp.where(feat_ok, pq, 0.0)
    pk = ratio * (jnp.exp(projk - diag_k - stab_k) + 1e-6)
    pk = jnp.where(feat_ok, pk, 0.0)

    kv = _dotbf(pk, v, ((0,), (0,)))
    num = _dotbf(pq, kv, ((1,), (0,)))
    pk_sum = jnp.sum(pk, axis=0, keepdims=True)
    den = jnp.sum(pq.astype(jnp.bfloat16).astype(f32)
                  * pk_sum.astype(jnp.bfloat16).astype(f32),
                  axis=-1, keepdims=True)
    ctx = num / (den + 1e-6)

    encw = encw_ref[...]
    h = (_dotbf(v, encw[0:DH], ((1,), (0,)))
         + _dotbf(ctx, encw[DH:2 * DH], ((1,), (0,)))
         + _dotbf(v * ctx, encw[2 * DH:3 * DH], ((1,), (0,)))
         + encb_ref[...])
    mu = jnp.mean(h, axis=-1, keepdims=True)
    var = jnp.mean((h - mu) * (h - mu), axis=-1, keepdims=True)
    h = (h - mu) * jax.lax.rsqrt(var + 1e-5) * lng_ref[...] + lnb_ref[...]
    h = jax.nn.gelu(h)
    lowres = _dotbf(h, decw_ref[...], ((1,), (0,))) + decb_ref[...]

    col = jax.lax.broadcasted_iota(jnp.int32, (T, PRED_LEN), 1)
    x = lowres
    m = jnp.max(x, axis=-1, keepdims=True)
    for _ in range(NSEL - 1):
        first = jnp.min(jnp.where(x >= m, col, PRED_LEN), axis=-1, keepdims=True)
        x = jnp.where(col == first, neg, x)
        m = jnp.max(x, axis=-1, keepdims=True)
    gmask = jnp.where(lowres >= m, 1.0, 0.0).astype(f32)

    expand = expand_ref[...]
    scale = DH ** -0.5
    for i in range(T // TQ):
        sl = slice(i * TQ, (i + 1) * TQ)
        s = _dotbf(q[sl], k, ((1,), (1,))) * scale
        mm = _dotbf(gmask[sl], expand, ((1,), (0,)))
        s = jnp.where(mm > 0.5, s, jnp.float32(-1e9))
        smax = jnp.max(s, axis=-1, keepdims=True)
        p = jnp.exp(s - smax)
        p = p / jnp.sum(p, axis=-1, keepdims=True)
        o_ref[0, 0, sl, :] = _dotbf(p, v, ((1,), (0,)))


@functools.partial(jax.jit, static_argnames=("interpret",))
def kernel(q, k, v, W_perf, enc_w, enc_b, ln_g, ln_b, dec_w, dec_b,
           interpret=False):
    f32 = jnp.float32
    w_pad = jnp.zeros((MPAD, DH), f32).at[:NB_FEAT].set(W_perf)
    encb = enc_b.reshape(1, 2 * DH)
    lng = ln_g.reshape(1, 2 * DH)
    lnb = ln_b.reshape(1, 2 * DH)
    decb = dec_b.reshape(1, PRED_LEN)

    gid_np = (np.arange(T) * PRED_LEN) // T
    expand = jnp.asarray(
        (gid_np[None, :] == np.arange(PRED_LEN)[:, None]).astype(np.float32))

    head_spec = pl.BlockSpec((1, 1, T, DH), lambda h: (0, h, 0, 0))
    full = lambda shape: pl.BlockSpec(shape, lambda h: tuple(0 for _ in shape))

    out = pl.pallas_call(
        _fused_kernel,
        grid=(H,),
        in_specs=[head_spec, head_spec, head_spec,
                  full((MPAD, DH)), full((3 * DH, 2 * DH)), full((1, 2 * DH)),
                  full((1, 2 * DH)), full((1, 2 * DH)),
                  full((2 * DH, PRED_LEN)), full((1, PRED_LEN)),
                  full((PRED_LEN, T))],
        out_specs=head_spec,
        out_shape=jax.ShapeDtypeStruct((B, H, T, DH), f32),
        compiler_params=pltpu.CompilerParams(
            dimension_semantics=("arbitrary",)),
        interpret=interpret,
    )(q, k, v, w_pad, enc_w, encb, lng, lnb, dec_w, decb, expand)

    return out
